# baseline (speedup 1.0000x reference)
import functools

import jax
import jax.numpy as jnp
from jax import lax
from jax.experimental import pallas as pl
from jax.experimental.pallas import tpu as pltpu
from jax.experimental.pallas import tpu_sc as plsc

DELTA_X = 0.01
INV_DX2 = 1.0 / (DELTA_X * DELTA_X)

NC = 2
NS = 16
NW = NC * NS


def _sc_scatter_sum(dst_idx, edge_attr, zeros):
    n_edges = dst_idx.shape[0]
    n_nodes, d = zeros.shape
    edges_per_w = n_edges // NW
    chunk = 80
    n_chunks = edges_per_w // chunk
    assert n_chunks * chunk == edges_per_w
    stripe = (n_nodes // NS) // 8 * 8
    tail_off = NS * stripe
    tail = n_nodes - tail_off

    mesh = plsc.VectorSubcoreMesh(core_axis_name="c", subcore_axis_name="s")

    nbuf = 4

    @functools.partial(
        pl.kernel,
        out_type=jax.ShapeDtypeStruct((NC, n_nodes, d), jnp.float32),
        mesh=mesh,
        scratch_types=[
            pltpu.VMEM_SHARED((n_nodes, d), jnp.float32),
            pltpu.VMEM((nbuf, chunk), jnp.int32),
            pltpu.VMEM((nbuf, chunk, d), jnp.float32),
            pltpu.SemaphoreType.DMA((nbuf,)),
            pltpu.SemaphoreType.DMA((nbuf,)),
            pltpu.SemaphoreType.DMA((nbuf,)),
        ],
    )
    def k(idx_hbm, attr_hbm, zeros_hbm, out_hbm, acc, idx_v, attr_v, fsem,
          isem, ssem):
        cid = lax.axis_index("c")
        sid = lax.axis_index("s")
        wid = sid * NC + cid

        r0 = sid * stripe
        pltpu.sync_copy(zeros_hbm.at[pl.ds(r0, stripe)],
                        acc.at[pl.ds(r0, stripe)])

        @pl.when(sid == NS - 1)
        def _():
            pltpu.sync_copy(zeros_hbm.at[pl.ds(tail_off, tail)],
                            acc.at[pl.ds(tail_off, tail)])

        plsc.subcore_barrier()

        base = wid * edges_per_w

        def attr_slice(i):
            return attr_hbm.at[pl.ds(base + i * chunk, chunk)]

        def idx_slice(i):
            return idx_hbm.at[pl.ds(base + i * chunk, chunk)]

        def start_fill(i, b):
            pltpu.async_copy(idx_slice(i), idx_v.at[b], isem.at[b])
            pltpu.async_copy(attr_slice(i), attr_v.at[b], fsem.at[b])

        def wait_fill(i, b):
            pltpu.make_async_copy(idx_slice(i), idx_v.at[b], isem.at[b]).wait()
            pltpu.make_async_copy(attr_slice(i), attr_v.at[b],
                                  fsem.at[b]).wait()

        def start_scatter(i, b):
            pltpu.async_copy(attr_v.at[b], acc.at[idx_v.at[b]], ssem.at[b],
                             add=True)

        def wait_scatter(b):
            pltpu.make_async_copy(attr_v.at[b], acc.at[idx_v.at[b]],
                                  ssem.at[b]).wait()

        for b in range(nbuf):
            start_fill(b, b)

        for i in range(2):
            wait_fill(i, i)
            start_scatter(i, i)
        for i in range(2, nbuf):
            wait_fill(i, i)
            start_scatter(i, i)
            wait_scatter(i - 2)
            start_fill(i + 2, i - 2)

        n_rounds = (n_chunks - 3) // nbuf

        @pl.loop(1, n_rounds)
        def _(r):
            for b in range(nbuf):
                i = r * nbuf + b
                wait_fill(i, b)
                start_scatter(i, b)
                wait_scatter((b + 2) % nbuf)
                start_fill(i + 2, (b + 2) % nbuf)

        for i in range(n_rounds * nbuf, n_chunks):
            b = i % nbuf
            wait_fill(i, b)
            start_scatter(i, b)
            wait_scatter((b + 2) % nbuf)
            if i + 2 < n_chunks:
                start_fill(i + 2, (b + 2) % nbuf)
        wait_scatter((n_chunks - 2) % nbuf)
        wait_scatter((n_chunks - 1) % nbuf)

        plsc.subcore_barrier()
        pltpu.sync_copy(acc.at[pl.ds(r0, stripe)],
                        out_hbm.at[cid, pl.ds(r0, stripe)])

        @pl.when(sid == NS - 1)
        def _():
            pltpu.sync_copy(acc.at[pl.ds(tail_off, tail)],
                            out_hbm.at[cid, pl.ds(tail_off, tail)])

    return k(dst_idx, edge_attr, zeros)


def _combine(partials, x):
    n_nodes, d = x.shape
    blk = 2000
    grid = n_nodes // blk

    def body(p_ref, x_ref, o_ref):
        o_ref[...] = (p_ref[0] + p_ref[1] - 2.0 * x_ref[...]) * INV_DX2

    return pl.pallas_call(
        body,
        grid=(grid,),
        in_specs=[
            pl.BlockSpec((NC, blk, d), lambda i: (0, i, 0)),
            pl.BlockSpec((blk, d), lambda i: (i, 0)),
        ],
        out_specs=pl.BlockSpec((blk, d), lambda i: (i, 0)),
        out_shape=jax.ShapeDtypeStruct((n_nodes, d), jnp.float32),
    )(partials, x)


def kernel(x, edge_index, edge_attr):
    dst = edge_index[1]
    if dst.dtype != jnp.int32:
        dst = dst.astype(jnp.int32)
    zeros = jnp.zeros(x.shape, jnp.float32)
    partials = _sc_scatter_sum(dst, edge_attr, zeros)
    return _combine(partials, x)


# speedup vs baseline: 1.0398x; 1.0398x over previous
import functools

import jax
import jax.numpy as jnp
from jax import lax
from jax.experimental import pallas as pl
from jax.experimental.pallas import tpu as pltpu
from jax.experimental.pallas import tpu_sc as plsc

DELTA_X = 0.01
INV_DX2 = 1.0 / (DELTA_X * DELTA_X)

NC = 2
NS = 16
NW = NC * NS


def _sc_scatter_sum(dst_idx, edge_attr, zeros):
    n_edges = dst_idx.shape[0]
    n_nodes, d = zeros.shape
    edges_per_w = n_edges // NW
    chunk = 128
    n_chunks = edges_per_w // chunk
    tail_e = edges_per_w - n_chunks * chunk
    stripe = (n_nodes // NS) // 8 * 8
    tail_off = NS * stripe
    tail = n_nodes - tail_off

    mesh = plsc.VectorSubcoreMesh(core_axis_name="c", subcore_axis_name="s")

    nbuf = 2
    rounds = n_chunks // nbuf
    tail_chunks = n_chunks - rounds * nbuf

    @functools.partial(
        pl.kernel,
        out_type=jax.ShapeDtypeStruct((NC, n_nodes, d), jnp.float32),
        mesh=mesh,
        scratch_types=[
            pltpu.VMEM_SHARED((n_nodes, d), jnp.float32),
            pltpu.VMEM((n_chunks, chunk), jnp.int32),
            pltpu.VMEM((1, tail_e), jnp.int32),
            pltpu.VMEM((nbuf, chunk, d), jnp.float32),
            pltpu.SemaphoreType.DMA((nbuf,)),
        ],
    )
    def k(idx_hbm, tidx_hbm, attr_hbm, zeros_hbm, out_hbm, acc, idx_v, tidx_v,
          attr_v, fsem):
        cid = lax.axis_index("c")
        sid = lax.axis_index("s")
        wid = sid * NC + cid

        r0 = sid * stripe
        pltpu.sync_copy(zeros_hbm.at[pl.ds(r0, stripe)],
                        acc.at[pl.ds(r0, stripe)])

        @pl.when(sid == NS - 1)
        def _():
            pltpu.sync_copy(zeros_hbm.at[pl.ds(tail_off, tail)],
                            acc.at[pl.ds(tail_off, tail)])

        plsc.subcore_barrier()

        base = wid * edges_per_w
        pltpu.sync_copy(idx_hbm.at[wid], idx_v)

        def attr_slice(i):
            return attr_hbm.at[pl.ds(base + i * chunk, chunk)]

        for b in range(nbuf):
            pltpu.async_copy(attr_slice(b), attr_v.at[b], fsem.at[b])

        @pl.loop(0, rounds)
        def _(r):
            for b in range(nbuf):
                i = r * nbuf + b
                pltpu.make_async_copy(attr_slice(i), attr_v.at[b],
                                      fsem.at[b]).wait()
                pltpu.sync_copy(attr_v.at[b], acc.at[idx_v.at[i]], add=True)
                nxt = i + nbuf

                @pl.when(nxt < n_chunks)
                def _():
                    pltpu.async_copy(attr_slice(nxt), attr_v.at[b], fsem.at[b])

        for t in range(tail_chunks):
            i = rounds * nbuf + t
            pltpu.make_async_copy(attr_slice(i), attr_v.at[t],
                                  fsem.at[t]).wait()
            pltpu.sync_copy(attr_v.at[t], acc.at[idx_v.at[i]], add=True)

        pltpu.sync_copy(tidx_hbm.at[wid], tidx_v)
        pltpu.sync_copy(attr_hbm.at[pl.ds(base + n_chunks * chunk, tail_e)],
                        attr_v.at[0, pl.ds(0, tail_e)])
        pltpu.sync_copy(attr_v.at[0, pl.ds(0, tail_e)],
                        acc.at[tidx_v.at[0]], add=True)

        plsc.subcore_barrier()
        pltpu.sync_copy(acc.at[pl.ds(r0, stripe)],
                        out_hbm.at[cid, pl.ds(r0, stripe)])

        @pl.when(sid == NS - 1)
        def _():
            pltpu.sync_copy(acc.at[pl.ds(tail_off, tail)],
                            out_hbm.at[cid, pl.ds(tail_off, tail)])

    dst2 = dst_idx.reshape(NW, edges_per_w)
    idx_main = dst2[:, :n_chunks * chunk].reshape(NW, n_chunks, chunk)
    idx_tail = dst2[:, n_chunks * chunk:].reshape(NW, 1, tail_e)
    return k(idx_main, idx_tail, edge_attr, zeros)


def _combine(partials, x):
    n_nodes, d = x.shape
    blk = 2000
    grid = n_nodes // blk

    def body(p_ref, x_ref, o_ref):
        o_ref[...] = (p_ref[0] + p_ref[1] - 2.0 * x_ref[...]) * INV_DX2

    return pl.pallas_call(
        body,
        grid=(grid,),
        in_specs=[
            pl.BlockSpec((NC, blk, d), lambda i: (0, i, 0)),
            pl.BlockSpec((blk, d), lambda i: (i, 0)),
        ],
        out_specs=pl.BlockSpec((blk, d), lambda i: (i, 0)),
        out_shape=jax.ShapeDtypeStruct((n_nodes, d), jnp.float32),
    )(partials, x)


def kernel(x, edge_index, edge_attr):
    dst = edge_index[1].astype(jnp.int32)
    zeros = jnp.zeros(x.shape, jnp.float32)
    partials = _sc_scatter_sum(dst, edge_attr, zeros)
    return _combine(partials, x)


# speedup vs baseline: 1.2529x; 1.2049x over previous
import functools

import jax
import jax.numpy as jnp
from jax import lax
from jax.experimental import pallas as pl
from jax.experimental.pallas import tpu as pltpu
from jax.experimental.pallas import tpu_sc as plsc

DELTA_X = 0.01
INV_DX2 = 1.0 / (DELTA_X * DELTA_X)

NC = 2
NS = 16
NW = NC * NS


def _sc_scatter_sum(edge_index4, edge_attr, n_nodes):
    n_chunks, chunk = edge_index4.shape[2], edge_index4.shape[3]
    d = edge_attr.shape[1]
    edges_per_w = n_chunks * chunk
    stripe = (n_nodes // NS) // 8 * 8
    tail_off = NS * stripe
    tail = n_nodes - tail_off

    mesh = plsc.VectorSubcoreMesh(core_axis_name="c", subcore_axis_name="s")

    nbuf = 3
    rounds = n_chunks // nbuf
    tail_chunks = n_chunks - rounds * nbuf
    z_copies = stripe // chunk
    z_rem = stripe - z_copies * chunk

    @functools.partial(
        pl.kernel,
        out_type=jax.ShapeDtypeStruct((NC, n_nodes, d), jnp.float32),
        mesh=mesh,
        scratch_types=[
            pltpu.VMEM_SHARED((n_nodes, d), jnp.float32),
            pltpu.VMEM((n_chunks, chunk), jnp.int32),
            pltpu.VMEM((nbuf, chunk, d), jnp.float32),
            pltpu.SemaphoreType.DMA((nbuf,)),
        ],
    )
    def k(idx_hbm, attr_hbm, out_hbm, acc, idx_v, attr_v, fsem):
        cid = lax.axis_index("c")
        sid = lax.axis_index("s")
        wid = sid * NC + cid

        zvec = jnp.zeros((16,), jnp.float32)
        for r in range(chunk):
            for c in range(0, d, 16):
                attr_v[0, r, pl.ds(c, 16)] = zvec
        r0 = sid * stripe
        for j in range(z_copies):
            pltpu.async_copy(attr_v.at[0], acc.at[pl.ds(r0 + j * chunk, chunk)],
                             fsem.at[0])
        pltpu.async_copy(attr_v.at[0, pl.ds(0, z_rem)],
                         acc.at[pl.ds(r0 + z_copies * chunk, z_rem)],
                         fsem.at[0])

        @pl.when(sid == NS - 1)
        def _():
            pltpu.async_copy(attr_v.at[0, pl.ds(0, tail)],
                             acc.at[pl.ds(tail_off, tail)], fsem.at[0])

        for j in range(z_copies):
            pltpu.make_async_copy(attr_v.at[0],
                                  acc.at[pl.ds(r0 + j * chunk, chunk)],
                                  fsem.at[0]).wait()
        pltpu.make_async_copy(attr_v.at[0, pl.ds(0, z_rem)],
                              acc.at[pl.ds(r0 + z_copies * chunk, z_rem)],
                              fsem.at[0]).wait()

        @pl.when(sid == NS - 1)
        def _():
            pltpu.make_async_copy(attr_v.at[0, pl.ds(0, tail)],
                                  acc.at[pl.ds(tail_off, tail)],
                                  fsem.at[0]).wait()

        plsc.subcore_barrier()

        base = wid * edges_per_w
        pltpu.sync_copy(idx_hbm.at[1, wid], idx_v)

        def attr_slice(i):
            return attr_hbm.at[pl.ds(base + i * chunk, chunk)]

        for b in range(nbuf):
            pltpu.async_copy(attr_slice(b), attr_v.at[b], fsem.at[b])

        @pl.loop(0, rounds)
        def _(r):
            for b in range(nbuf):
                i = r * nbuf + b
                pltpu.make_async_copy(attr_slice(i), attr_v.at[b],
                                      fsem.at[b]).wait()
                pltpu.sync_copy(attr_v.at[b], acc.at[idx_v.at[i]], add=True)
                nxt = i + nbuf

                @pl.when(nxt < n_chunks)
                def _():
                    pltpu.async_copy(attr_slice(nxt), attr_v.at[b], fsem.at[b])

        for t in range(tail_chunks):
            i = rounds * nbuf + t
            pltpu.make_async_copy(attr_slice(i), attr_v.at[t],
                                  fsem.at[t]).wait()
            pltpu.sync_copy(attr_v.at[t], acc.at[idx_v.at[i]], add=True)

        plsc.subcore_barrier()
        pltpu.sync_copy(acc.at[pl.ds(r0, stripe)],
                        out_hbm.at[cid, pl.ds(r0, stripe)])

        @pl.when(sid == NS - 1)
        def _():
            pltpu.sync_copy(acc.at[pl.ds(tail_off, tail)],
                            out_hbm.at[cid, pl.ds(tail_off, tail)])

    return k(edge_index4, edge_attr)


def _combine(partials, x):
    n_nodes, d = x.shape
    blk = 2000
    grid = n_nodes // blk

    def body(p_ref, x_ref, o_ref):
        o_ref[...] = (p_ref[0] + p_ref[1] - 2.0 * x_ref[...]) * INV_DX2

    return pl.pallas_call(
        body,
        grid=(grid,),
        in_specs=[
            pl.BlockSpec((NC, blk, d), lambda i: (0, i, 0)),
            pl.BlockSpec((blk, d), lambda i: (i, 0)),
        ],
        out_specs=pl.BlockSpec((blk, d), lambda i: (i, 0)),
        out_shape=jax.ShapeDtypeStruct((n_nodes, d), jnp.float32),
    )(partials, x)


def kernel(x, edge_index, edge_attr):
    if edge_index.dtype != jnp.int32:
        edge_index = edge_index.astype(jnp.int32)
    n_edges = edge_index.shape[1]
    chunk = 80
    edge_index4 = edge_index.reshape(2, NW, n_edges // (NW * chunk), chunk)
    partials = _sc_scatter_sum(edge_index4, edge_attr, x.shape[0])
    return _combine(partials, x)


# speedup vs baseline: 1.2562x; 1.0027x over previous
import functools

import jax
import jax.numpy as jnp
from jax import lax
from jax.experimental import pallas as pl
from jax.experimental.pallas import tpu as pltpu
from jax.experimental.pallas import tpu_sc as plsc

DELTA_X = 0.01
INV_DX2 = 1.0 / (DELTA_X * DELTA_X)

NC = 2
NS = 16
NW = NC * NS


def _sc_scatter_sum(edge_index4, edge_attr, n_nodes):
    n_chunks, chunk = edge_index4.shape[2], edge_index4.shape[3]
    d = edge_attr.shape[1]
    edges_per_w = n_chunks * chunk
    stripe = (n_nodes // NS) // 8 * 8
    tail_off = NS * stripe
    tail = n_nodes - tail_off

    mesh = plsc.VectorSubcoreMesh(core_axis_name="c", subcore_axis_name="s")

    nbuf = 3
    rounds = n_chunks // nbuf
    tail_chunks = n_chunks - rounds * nbuf
    zb = 16
    z_copies = stripe // zb
    assert z_copies * zb == stripe

    @functools.partial(
        pl.kernel,
        out_type=jax.ShapeDtypeStruct((NC, n_nodes, d), jnp.float32),
        mesh=mesh,
        scratch_types=[
            pltpu.VMEM_SHARED((n_nodes, d), jnp.float32),
            pltpu.VMEM((n_chunks, chunk), jnp.int32),
            pltpu.VMEM((nbuf, chunk, d), jnp.float32),
            pltpu.SemaphoreType.DMA((nbuf,)),
        ],
    )
    def k(idx_hbm, attr_hbm, out_hbm, acc, idx_v, attr_v, fsem):
        cid = lax.axis_index("c")
        sid = lax.axis_index("s")
        wid = sid * NC + cid

        zvec = jnp.zeros((16,), jnp.float32)
        for r in range(zb):
            for c in range(0, d, 16):
                attr_v[0, r, pl.ds(c, 16)] = zvec
        r0 = sid * stripe
        zsrc = attr_v.at[0, pl.ds(0, zb)]

        zg = 4
        zrem = z_copies % zg

        @pl.loop(0, z_copies // zg)
        def _(g):
            for u in range(zg):
                pltpu.async_copy(zsrc, acc.at[pl.ds(r0 + (g * zg + u) * zb,
                                                    zb)], fsem.at[0])
            for u in range(zg):
                pltpu.make_async_copy(zsrc,
                                      acc.at[pl.ds(r0 + (g * zg + u) * zb,
                                                   zb)], fsem.at[0]).wait()

        for u in range(zrem):
            j = z_copies - zrem + u
            pltpu.async_copy(zsrc, acc.at[pl.ds(r0 + j * zb, zb)], fsem.at[0])

        @pl.when(sid == NS - 1)
        def _():
            pltpu.async_copy(zsrc, acc.at[pl.ds(tail_off, tail)], fsem.at[0])

        for u in range(zrem):
            j = z_copies - zrem + u
            pltpu.make_async_copy(zsrc, acc.at[pl.ds(r0 + j * zb, zb)],
                                  fsem.at[0]).wait()

        @pl.when(sid == NS - 1)
        def _():
            pltpu.make_async_copy(zsrc, acc.at[pl.ds(tail_off, tail)],
                                  fsem.at[0]).wait()

        plsc.subcore_barrier()

        base = wid * edges_per_w
        pltpu.sync_copy(idx_hbm.at[1, wid], idx_v)

        def attr_slice(i):
            return attr_hbm.at[pl.ds(base + i * chunk, chunk)]

        for b in range(nbuf):
            pltpu.async_copy(attr_slice(b), attr_v.at[b], fsem.at[b])

        @pl.loop(0, rounds)
        def _(r):
            for b in range(nbuf):
                i = r * nbuf + b
                pltpu.make_async_copy(attr_slice(i), attr_v.at[b],
                                      fsem.at[b]).wait()
                pltpu.sync_copy(attr_v.at[b], acc.at[idx_v.at[i]], add=True)
                nxt = i + nbuf

                @pl.when(nxt < n_chunks)
                def _():
                    pltpu.async_copy(attr_slice(nxt), attr_v.at[b], fsem.at[b])

        for t in range(tail_chunks):
            i = rounds * nbuf + t
            pltpu.make_async_copy(attr_slice(i), attr_v.at[t],
                                  fsem.at[t]).wait()
            pltpu.sync_copy(attr_v.at[t], acc.at[idx_v.at[i]], add=True)

        plsc.subcore_barrier()
        pltpu.sync_copy(acc.at[pl.ds(r0, stripe)],
                        out_hbm.at[cid, pl.ds(r0, stripe)])

        @pl.when(sid == NS - 1)
        def _():
            pltpu.sync_copy(acc.at[pl.ds(tail_off, tail)],
                            out_hbm.at[cid, pl.ds(tail_off, tail)])

    return k(edge_index4, edge_attr)


def _combine(partials, x):
    n_nodes, d = x.shape
    blk = 2000
    grid = n_nodes // blk

    def body(p_ref, x_ref, o_ref):
        o_ref[...] = (p_ref[0] + p_ref[1] - 2.0 * x_ref[...]) * INV_DX2

    return pl.pallas_call(
        body,
        grid=(grid,),
        in_specs=[
            pl.BlockSpec((NC, blk, d), lambda i: (0, i, 0)),
            pl.BlockSpec((blk, d), lambda i: (i, 0)),
        ],
        out_specs=pl.BlockSpec((blk, d), lambda i: (i, 0)),
        out_shape=jax.ShapeDtypeStruct((n_nodes, d), jnp.float32),
    )(partials, x)


def kernel(x, edge_index, edge_attr):
    if edge_index.dtype != jnp.int32:
        edge_index = edge_index.astype(jnp.int32)
    n_edges = edge_index.shape[1]
    chunk = 80
    edge_index4 = edge_index.reshape(2, NW, n_edges // (NW * chunk), chunk)
    partials = _sc_scatter_sum(edge_index4, edge_attr, x.shape[0])
    return _combine(partials, x)


# speedup vs baseline: 1.2909x; 1.0276x over previous
import functools

import jax
import jax.numpy as jnp
from jax import lax
from jax.experimental import pallas as pl
from jax.experimental.pallas import tpu as pltpu
from jax.experimental.pallas import tpu_sc as plsc

DELTA_X = 0.01
INV_DX2 = 1.0 / (DELTA_X * DELTA_X)

NC = 2
NS = 16
NW = NC * NS


def _sc_scatter_sum(edge_index4, edge_attr, n_nodes):
    n_chunks, chunk = edge_index4.shape[2], edge_index4.shape[3]
    d = edge_attr.shape[1]
    edges_per_w = n_chunks * chunk
    stripe = (n_nodes // NS) // 8 * 8
    tail_off = NS * stripe
    tail = n_nodes - tail_off

    mesh = plsc.VectorSubcoreMesh(core_axis_name="c", subcore_axis_name="s")

    nbuf = 3
    rounds = n_chunks // nbuf
    tail_chunks = n_chunks - rounds * nbuf
    zb = 16
    z_copies = stripe // zb
    assert z_copies * zb == stripe

    @functools.partial(
        pl.kernel,
        out_type=jax.ShapeDtypeStruct((NC, n_nodes, d), jnp.float32),
        mesh=mesh,
        scratch_types=[
            pltpu.VMEM_SHARED((n_nodes, d), jnp.float32),
            pltpu.VMEM((n_chunks, chunk), jnp.int32),
            pltpu.VMEM((nbuf, chunk, d), jnp.float32),
            pltpu.VMEM((zb, d), jnp.float32),
            pltpu.SemaphoreType.DMA((nbuf,)),
            pltpu.SemaphoreType.DMA,
            pltpu.SemaphoreType.DMA,
        ],
    )
    def k(idx_hbm, attr_hbm, out_hbm, acc, idx_v, attr_v, zbuf, fsem, zsem,
          isem):
        cid = lax.axis_index("c")
        sid = lax.axis_index("s")
        wid = sid * NC + cid
        base = wid * edges_per_w

        def attr_slice(i):
            return attr_hbm.at[pl.ds(base + i * chunk, chunk)]

        for b in range(nbuf):
            pltpu.async_copy(attr_slice(b), attr_v.at[b], fsem.at[b])
        pltpu.async_copy(idx_hbm.at[1, wid], idx_v, isem)

        zvec = jnp.zeros((16,), jnp.float32)
        for r in range(zb):
            for c in range(0, d, 16):
                zbuf[r, pl.ds(c, 16)] = zvec
        r0 = sid * stripe
        zsrc = zbuf.at[pl.ds(0, zb)]

        zg = 4
        zrem = z_copies % zg

        @pl.loop(0, z_copies // zg)
        def _(g):
            for u in range(zg):
                pltpu.async_copy(zsrc, acc.at[pl.ds(r0 + (g * zg + u) * zb,
                                                    zb)], zsem)
            for u in range(zg):
                pltpu.make_async_copy(zsrc,
                                      acc.at[pl.ds(r0 + (g * zg + u) * zb,
                                                   zb)], zsem).wait()

        for u in range(zrem):
            j = z_copies - zrem + u
            pltpu.async_copy(zsrc, acc.at[pl.ds(r0 + j * zb, zb)], zsem)

        @pl.when(sid == NS - 1)
        def _():
            pltpu.async_copy(zsrc, acc.at[pl.ds(tail_off, tail)], zsem)

        for u in range(zrem):
            j = z_copies - zrem + u
            pltpu.make_async_copy(zsrc, acc.at[pl.ds(r0 + j * zb, zb)],
                                  zsem).wait()

        @pl.when(sid == NS - 1)
        def _():
            pltpu.make_async_copy(zsrc, acc.at[pl.ds(tail_off, tail)],
                                  zsem).wait()

        plsc.subcore_barrier()
        pltpu.make_async_copy(idx_hbm.at[1, wid], idx_v, isem).wait()

        @pl.loop(0, rounds)
        def _(r):
            for b in range(nbuf):
                i = r * nbuf + b
                pltpu.make_async_copy(attr_slice(i), attr_v.at[b],
                                      fsem.at[b]).wait()
                pltpu.sync_copy(attr_v.at[b], acc.at[idx_v.at[i]], add=True)
                nxt = i + nbuf

                @pl.when(nxt < n_chunks)
                def _():
                    pltpu.async_copy(attr_slice(nxt), attr_v.at[b], fsem.at[b])

        for t in range(tail_chunks):
            i = rounds * nbuf + t
            pltpu.make_async_copy(attr_slice(i), attr_v.at[t],
                                  fsem.at[t]).wait()
            pltpu.sync_copy(attr_v.at[t], acc.at[idx_v.at[i]], add=True)

        plsc.subcore_barrier()
        pltpu.sync_copy(acc.at[pl.ds(r0, stripe)],
                        out_hbm.at[cid, pl.ds(r0, stripe)])

        @pl.when(sid == NS - 1)
        def _():
            pltpu.sync_copy(acc.at[pl.ds(tail_off, tail)],
                            out_hbm.at[cid, pl.ds(tail_off, tail)])

    return k(edge_index4, edge_attr)


def _combine(partials, x):
    n_nodes, d = x.shape
    blk = 2000
    grid = n_nodes // blk

    def body(p_ref, x_ref, o_ref):
        o_ref[...] = (p_ref[0] + p_ref[1] - 2.0 * x_ref[...]) * INV_DX2

    return pl.pallas_call(
        body,
        grid=(grid,),
        in_specs=[
            pl.BlockSpec((NC, blk, d), lambda i: (0, i, 0)),
            pl.BlockSpec((blk, d), lambda i: (i, 0)),
        ],
        out_specs=pl.BlockSpec((blk, d), lambda i: (i, 0)),
        out_shape=jax.ShapeDtypeStruct((n_nodes, d), jnp.float32),
    )(partials, x)


def kernel(x, edge_index, edge_attr):
    if edge_index.dtype != jnp.int32:
        edge_index = edge_index.astype(jnp.int32)
    n_edges = edge_index.shape[1]
    chunk = 80
    edge_index4 = edge_index.reshape(2, NW, n_edges // (NW * chunk), chunk)
    partials = _sc_scatter_sum(edge_index4, edge_attr, x.shape[0])
    return _combine(partials, x)
